# Initial kernel scaffold; baseline (speedup 1.0000x reference)
import jax
import jax.numpy as jnp
from jax.experimental import pallas as pl


def kernel(diagrams, variance):
    raise NotImplementedError("write your pallas kernel here")


# speedup vs baseline: 1.0606x; 1.0606x over previous
import math

import jax
import jax.numpy as jnp
from jax import lax
from jax.experimental import pallas as pl

N_PTS = 4096
BLK = 128
NY = 64
NX = 64
INV_STEP = 1.0 / 64.0


def _phi_body(var_ref, d_ref, out_ref):
    var = var_ref[0, 0]
    inv2s2 = 1.0 / (2.0 * var * var)
    norm = 1.0 / (2.0 * math.pi * var * var)

    b = d_ref[:, 0]
    p = d_ref[:, 1] - d_ref[:, 0]

    xv = lax.broadcasted_iota(jnp.int32, (BLK, NX), 1).astype(jnp.float32) * INV_STEP
    yv = lax.broadcasted_iota(jnp.int32, (BLK, NY), 1).astype(jnp.float32) * INV_STEP

    gx = jnp.exp(-jnp.square(b[:, None] - xv) * inv2s2) * norm
    gy = jnp.exp(-jnp.square(p[:, None] - yv) * inv2s2)

    out_ref[...] = gy[:, :, None] * gx[:, None, :]


def kernel(diagrams, variance):
    d = diagrams.reshape(N_PTS, 2)
    var = jnp.reshape(variance, (1, 1)).astype(jnp.float32)

    out = pl.pallas_call(
        _phi_body,
        grid=(N_PTS // BLK,),
        in_specs=[
            pl.BlockSpec((1, 1), lambda m: (0, 0)),
            pl.BlockSpec((BLK, 2), lambda m: (m, 0)),
        ],
        out_specs=pl.BlockSpec((BLK, NY, NX), lambda m: (m, 0, 0)),
        out_shape=jax.ShapeDtypeStruct((N_PTS, NY, NX), jnp.float32),
    )(var, d)

    return out.reshape(diagrams.shape[0], diagrams.shape[1], NY, NX, 1)
